# baseline (speedup 1.0000x reference)
import functools

import jax
import jax.numpy as jnp
from jax.experimental import pallas as pl


def _layer_kernel(adj_ref, m_ref, w_ref, b_ref, out_ref, *, relu):
    g = jnp.dot(adj_ref[...], m_ref[...], preferred_element_type=jnp.float32)
    h = jnp.dot(g, w_ref[...], preferred_element_type=jnp.float32) + b_ref[...]
    if relu:
        h = jnp.maximum(h, 0.0)
    out_ref[...] = h


def _layer(adj, m, w, b, relu, bm):
    n = adj.shape[0]
    d = m.shape[1]
    return pl.pallas_call(
        functools.partial(_layer_kernel, relu=relu),
        grid=(n // bm,),
        in_specs=[
            pl.BlockSpec((bm, n), lambda i: (i, 0)),
            pl.BlockSpec((n, d), lambda i: (0, 0)),
            pl.BlockSpec((d, d), lambda i: (0, 0)),
            pl.BlockSpec((1, d), lambda i: (0, 0)),
        ],
        out_specs=pl.BlockSpec((bm, d), lambda i: (i, 0)),
        out_shape=jax.ShapeDtypeStruct((n, d), jnp.float32),
    )(adj, m, w, b)


def kernel(x, adj, W1, b1, W2, b2):
    b1r = b1.reshape(1, -1)
    b2r = b2.reshape(1, -1)
    h = _layer(adj, x, W1, b1r, relu=True, bm=200)
    return _layer(adj, h, W2, b2r, relu=False, bm=200)


# speedup vs baseline: 1.0565x; 1.0565x over previous
import jax
import jax.numpy as jnp
from jax.experimental import pallas as pl
from jax.experimental.pallas import tpu as pltpu

_BM = 400


def _gcn_kernel(adj_ref, x_ref, w1_ref, b1_ref, w2_ref, b2_ref, out_ref,
                h_ref):
    p = pl.program_id(0)
    i = pl.program_id(1)

    @pl.when(p == 0)
    def _layer1():
        g = jnp.dot(adj_ref[...], x_ref[...],
                    preferred_element_type=jnp.float32)
        h = jnp.dot(g, w1_ref[...],
                    preferred_element_type=jnp.float32) + b1_ref[...]
        h_ref[pl.ds(i * _BM, _BM), :] = jnp.maximum(h, 0.0)

    @pl.when(p == 1)
    def _layer2():
        g = jnp.dot(adj_ref[...], h_ref[...],
                    preferred_element_type=jnp.float32)
        out_ref[...] = jnp.dot(g, w2_ref[...],
                               preferred_element_type=jnp.float32) + b2_ref[...]


def kernel(x, adj, W1, b1, W2, b2):
    n, d = x.shape
    nb = n // _BM
    return pl.pallas_call(
        _gcn_kernel,
        grid=(2, nb),
        in_specs=[
            pl.BlockSpec((_BM, n), lambda p, i: (i, 0)),
            pl.BlockSpec((n, d), lambda p, i: (0, 0)),
            pl.BlockSpec((d, d), lambda p, i: (0, 0)),
            pl.BlockSpec((1, d), lambda p, i: (0, 0)),
            pl.BlockSpec((d, d), lambda p, i: (0, 0)),
            pl.BlockSpec((1, d), lambda p, i: (0, 0)),
        ],
        out_specs=pl.BlockSpec((_BM, d), lambda p, i: (i * p, 0)),
        out_shape=jax.ShapeDtypeStruct((n, d), jnp.float32),
        scratch_shapes=[pltpu.VMEM((n, d), jnp.float32)],
    )(adj, x, W1, b1.reshape(1, -1), W2, b2.reshape(1, -1))
